# Initial kernel scaffold; baseline (speedup 1.0000x reference)
import jax
import jax.numpy as jnp
from jax.experimental import pallas as pl


def kernel(center_heatmap_pred, wh_pred, offset_pred, yaw_class_pred, yaw_res_pred, velocity_pred):
    raise NotImplementedError("write your pallas kernel here")


# speedup vs baseline: 1.0557x; 1.0557x over previous
import jax
import jax.numpy as jnp
import numpy as np
from jax.experimental import pallas as pl
from jax.experimental.pallas import tpu as pltpu

_NUM_CLASSES = 8
_NUM_DIR_BINS = 12
_TOP_K = 100
_PPM = 4.0


def _nms_body(h_ref, o_ref):
    x = h_ref[0]
    neg = jnp.float32(-np.inf)
    pad_w = jnp.full_like(x[:, :, :1], neg)
    l = jnp.concatenate([pad_w, x[:, :, :-1]], axis=2)
    r = jnp.concatenate([x[:, :, 1:], pad_w], axis=2)
    hm = jnp.maximum(jnp.maximum(l, r), x)
    pad_h = jnp.full_like(hm[:, :1, :], neg)
    u = jnp.concatenate([pad_h, hm[:, :-1, :]], axis=1)
    d = jnp.concatenate([hm[:, 1:, :], pad_h], axis=1)
    vm = jnp.maximum(jnp.maximum(u, d), hm)
    o_ref[0] = jnp.where(vm == x, x, 0.0)


def _nms(heat):
    B, C, H, W = heat.shape
    return pl.pallas_call(
        _nms_body,
        out_shape=jax.ShapeDtypeStruct((B, C, H, W), jnp.float32),
        grid=(B,),
        in_specs=[pl.BlockSpec((1, C, H, W), lambda b: (b, 0, 0, 0))],
        out_specs=pl.BlockSpec((1, C, H, W), lambda b: (b, 0, 0, 0)),
    )(heat)


def _gather_feat(feat, ind):
    B, C, H, W = feat.shape
    f = jnp.transpose(feat, (0, 2, 3, 1)).reshape(B, H * W, C)
    idx = jnp.broadcast_to(ind[:, :, None], (B, ind.shape[1], C))
    return jnp.take_along_axis(f, idx, axis=1)


def kernel(center_heatmap_pred, wh_pred, offset_pred, yaw_class_pred,
           yaw_res_pred, velocity_pred):
    heat = _nms(center_heatmap_pred)
    B, C, H, W = heat.shape
    scores_flat = heat.reshape(B, -1)
    topk_scores, topk_inds = jax.lax.top_k(scores_flat, _TOP_K)
    topk_clses = topk_inds // (H * W)
    topk_inds = topk_inds % (H * W)
    topk_ys = (topk_inds // W).astype(jnp.float32)
    topk_xs = (topk_inds % W).astype(jnp.float32)

    wh = _gather_feat(wh_pred, topk_inds)
    offset = _gather_feat(offset_pred, topk_inds)
    yaw_class_feat = _gather_feat(yaw_class_pred, topk_inds)
    yaw_res = _gather_feat(yaw_res_pred, topk_inds)

    yaw_class = jnp.argmax(yaw_class_feat, axis=-1)
    angle_per_class = 2.0 * np.pi / _NUM_DIR_BINS
    yaw = yaw_class.astype(jnp.float32) * angle_per_class + yaw_res[..., 0]
    brake = jnp.zeros_like(yaw)
    velocity = _gather_feat(velocity_pred, topk_inds)[..., 0]

    topk_xs = topk_xs + offset[..., 0]
    topk_ys = topk_ys + offset[..., 1]

    batch_bboxes = jnp.stack(
        [topk_xs, topk_ys, wh[..., 0], wh[..., 1], yaw, velocity, brake], axis=2)
    batch_bboxes = jnp.concatenate(
        [batch_bboxes,
         topk_clses[..., None].astype(jnp.float32),
         topk_scores[..., None]], axis=-1)
    batch_bboxes = batch_bboxes.at[:, :, :4].multiply(_PPM)
    return batch_bboxes


# speedup vs baseline: 2.6708x; 2.5299x over previous
import numpy as np

import jax
import jax.numpy as jnp
from jax import lax
from jax.experimental import pallas as pl
from jax.experimental.pallas import tpu as pltpu

_NUM_CLASSES = 8
_NUM_DIR_BINS = 12
_TOP_K = 100
_PPM = 4.0
_KP = 128
_BIG = np.int32(0x7FFFFFFF)


def _topk_body(h_ref, outv_ref, outi_ref, s_ref, l1v_ref, l1i_ref,
               l2v_ref, l2i_ref):
    x = h_ref[0]
    neg = jnp.float32(-np.inf)
    pad_w = jnp.full_like(x[:, :, :1], neg)
    left = jnp.concatenate([pad_w, x[:, :, :-1]], axis=2)
    right = jnp.concatenate([x[:, :, 1:], pad_w], axis=2)
    hm = jnp.maximum(jnp.maximum(left, right), x)
    pad_h = jnp.full_like(hm[:, :1, :], neg)
    up = jnp.concatenate([pad_h, hm[:, :-1, :]], axis=1)
    dn = jnp.concatenate([hm[:, 1:, :], pad_h], axis=1)
    vm = jnp.maximum(jnp.maximum(up, dn), hm)
    s = jnp.where(vm == x, x, 0.0).reshape(2048, 256)
    s_ref[...] = s

    fi = (lax.broadcasted_iota(jnp.int32, (2048, 256), 0) * 256
          + lax.broadcasted_iota(jnp.int32, (2048, 256), 1))
    s3 = s.reshape(128, 16, 256)
    fi3 = fi.reshape(128, 16, 256)
    l1v = jnp.max(s3, axis=1)
    l1i = jnp.min(jnp.where(s3 == l1v[:, None, :], fi3, _BIG), axis=1)
    l1v_ref[...] = l1v
    l1i_ref[...] = l1i
    l1v3 = l1v.reshape(16, 8, 256)
    l1i3 = l1i.reshape(16, 8, 256)
    l2v = jnp.max(l1v3, axis=1)
    l2i = jnp.min(jnp.where(l1v3 == l2v[:, None, :], l1i3, _BIG), axis=1)
    l2v_ref[...] = l2v
    l2i_ref[...] = l2i

    outv_ref[...] = jnp.zeros((1, 1, _KP), jnp.float32)
    outi_ref[...] = jnp.zeros((1, 1, _KP), jnp.int32)
    lane = lax.broadcasted_iota(jnp.int32, (1, _KP), 1)
    colio = lax.broadcasted_iota(jnp.int32, (1, 256), 1)
    bio = (lax.broadcasted_iota(jnp.int32, (16, 256), 0) * 256
           + lax.broadcasted_iota(jnp.int32, (16, 256), 1))

    def body(k, _):
        l2v_ = l2v_ref[...]
        l2i_ = l2i_ref[...]
        l3v = jnp.max(l2v_, axis=0, keepdims=True)
        l3i = jnp.min(jnp.where(l2v_ == l3v, l2i_, _BIG), axis=0,
                      keepdims=True)
        m = jnp.max(l3v)
        f = jnp.min(jnp.where(l3v == m, l3i, _BIG))
        outv_ref[0] = jnp.where(lane == k, m, outv_ref[0])
        outi_ref[0] = jnp.where(lane == k, f, outi_ref[0])
        row = lax.shift_right_logical(f, 8)
        col = jnp.bitwise_and(f, 255)
        srow = s_ref[pl.ds(row, 1), :]
        s_ref[pl.ds(row, 1), :] = jnp.where(colio == col, -1.0, srow)
        g = lax.shift_right_logical(row, 4)
        blk = s_ref[pl.ds(g * 16, 16), :]
        bfi = bio + g * 4096
        bm = jnp.max(blk, axis=0, keepdims=True)
        bi = jnp.min(jnp.where(blk == bm, bfi, _BIG), axis=0, keepdims=True)
        l1v_ref[pl.ds(g, 1), :] = bm
        l1i_ref[pl.ds(g, 1), :] = bi
        g2 = lax.shift_right_logical(g, 3)
        b2v = l1v_ref[pl.ds(g2 * 8, 8), :]
        b2i = l1i_ref[pl.ds(g2 * 8, 8), :]
        b2m = jnp.max(b2v, axis=0, keepdims=True)
        b2mi = jnp.min(jnp.where(b2v == b2m, b2i, _BIG), axis=0,
                       keepdims=True)
        l2v_ref[pl.ds(g2, 1), :] = b2m
        l2i_ref[pl.ds(g2, 1), :] = b2mi
        return 0

    lax.fori_loop(0, _TOP_K, body, 0)


def _topk_tc(heat, interpret=False):
    B, C, H, W = heat.shape
    return pl.pallas_call(
        _topk_body,
        out_shape=[jax.ShapeDtypeStruct((B, 1, _KP), jnp.float32),
                   jax.ShapeDtypeStruct((B, 1, _KP), jnp.int32)],
        grid=(B,),
        in_specs=[pl.BlockSpec((1, C, H, W), lambda b: (b, 0, 0, 0))],
        out_specs=[pl.BlockSpec((1, 1, _KP), lambda b: (b, 0, 0)),
                   pl.BlockSpec((1, 1, _KP), lambda b: (b, 0, 0))],
        scratch_shapes=[
            pltpu.VMEM((2048, 256), jnp.float32),
            pltpu.VMEM((128, 256), jnp.float32),
            pltpu.VMEM((128, 256), jnp.int32),
            pltpu.VMEM((16, 256), jnp.float32),
            pltpu.VMEM((16, 256), jnp.int32),
        ],
        interpret=interpret,
    )(heat)


def _gather_feat(feat, ind):
    B, C, H, W = feat.shape
    f = jnp.transpose(feat, (0, 2, 3, 1)).reshape(B, H * W, C)
    idx = jnp.broadcast_to(ind[:, :, None], (B, ind.shape[1], C))
    return jnp.take_along_axis(f, idx, axis=1)


def kernel(center_heatmap_pred, wh_pred, offset_pred, yaw_class_pred,
           yaw_res_pred, velocity_pred):
    B, C, H, W = center_heatmap_pred.shape
    scores_p, inds_p = _topk_tc(center_heatmap_pred)
    topk_scores = scores_p.reshape(B, _KP)[:, :_TOP_K]
    flat_inds = inds_p.reshape(B, _KP)[:, :_TOP_K]
    topk_clses = flat_inds // (H * W)
    topk_inds = flat_inds % (H * W)
    topk_ys = (topk_inds // W).astype(jnp.float32)
    topk_xs = (topk_inds % W).astype(jnp.float32)

    wh = _gather_feat(wh_pred, topk_inds)
    offset = _gather_feat(offset_pred, topk_inds)
    yaw_class_feat = _gather_feat(yaw_class_pred, topk_inds)
    yaw_res = _gather_feat(yaw_res_pred, topk_inds)

    yaw_class = jnp.argmax(yaw_class_feat, axis=-1)
    angle_per_class = 2.0 * np.pi / _NUM_DIR_BINS
    yaw = yaw_class.astype(jnp.float32) * angle_per_class + yaw_res[..., 0]
    brake = jnp.zeros_like(yaw)
    velocity = _gather_feat(velocity_pred, topk_inds)[..., 0]

    topk_xs = topk_xs + offset[..., 0]
    topk_ys = topk_ys + offset[..., 1]

    batch_bboxes = jnp.stack(
        [topk_xs, topk_ys, wh[..., 0], wh[..., 1], yaw, velocity, brake],
        axis=2)
    batch_bboxes = jnp.concatenate(
        [batch_bboxes,
         topk_clses[..., None].astype(jnp.float32),
         topk_scores[..., None]], axis=-1)
    batch_bboxes = batch_bboxes.at[:, :, :4].multiply(_PPM)
    return batch_bboxes


# speedup vs baseline: 3.1023x; 1.1616x over previous
import numpy as np

import jax
import jax.numpy as jnp
from jax import lax
from jax.experimental import pallas as pl
from jax.experimental.pallas import tpu as pltpu

_NUM_CLASSES = 8
_NUM_DIR_BINS = 12
_TOP_K = 100
_PPM = 4.0
_KP = 128
_BIG = np.int32(0x7FFFFFFF)


def _topk_body(h_ref, outv_ref, outi_ref, s_ref, l1v_ref, l1i_ref,
               l2v_ref, l2i_ref):
    x = h_ref[0]
    neg = jnp.float32(-np.inf)
    pad_w = jnp.full_like(x[:, :, :1], neg)
    left = jnp.concatenate([pad_w, x[:, :, :-1]], axis=2)
    right = jnp.concatenate([x[:, :, 1:], pad_w], axis=2)
    hm = jnp.maximum(jnp.maximum(left, right), x)
    pad_h = jnp.full_like(hm[:, :1, :], neg)
    up = jnp.concatenate([pad_h, hm[:, :-1, :]], axis=1)
    dn = jnp.concatenate([hm[:, 1:, :], pad_h], axis=1)
    vm = jnp.maximum(jnp.maximum(up, dn), hm)
    s = jnp.where(vm == x, x, 0.0).reshape(2048, 256)
    s_ref[...] = s

    fi = (lax.broadcasted_iota(jnp.int32, (2048, 256), 0) * 256
          + lax.broadcasted_iota(jnp.int32, (2048, 256), 1))
    s3 = s.reshape(128, 16, 256)
    fi3 = fi.reshape(128, 16, 256)
    l1v = jnp.max(s3, axis=1)
    l1i = jnp.min(jnp.where(s3 == l1v[:, None, :], fi3, _BIG), axis=1)
    l1v_ref[...] = l1v
    l1i_ref[...] = l1i
    l1v3 = l1v.reshape(16, 8, 256)
    l1i3 = l1i.reshape(16, 8, 256)
    l2v = jnp.max(l1v3, axis=1)
    l2i = jnp.min(jnp.where(l1v3 == l2v[:, None, :], l1i3, _BIG), axis=1)
    l2v_ref[...] = l2v
    l2i_ref[...] = l2i

    outv_ref[...] = jnp.zeros((1, 1, _KP), jnp.float32)
    outi_ref[...] = jnp.zeros((1, 1, _KP), jnp.int32)
    lane = lax.broadcasted_iota(jnp.int32, (1, _KP), 1)
    colio = lax.broadcasted_iota(jnp.int32, (1, 256), 1)
    bio = (lax.broadcasted_iota(jnp.int32, (16, 256), 0) * 256
           + lax.broadcasted_iota(jnp.int32, (16, 256), 1))

    def body(k, _):
        l2v_ = l2v_ref[...]
        l2i_ = l2i_ref[...]
        l3v = jnp.max(l2v_, axis=0, keepdims=True)
        l3i = jnp.min(jnp.where(l2v_ == l3v, l2i_, _BIG), axis=0,
                      keepdims=True)
        m = jnp.max(l3v)
        f = jnp.min(jnp.where(l3v == m, l3i, _BIG))
        outv_ref[0] = jnp.where(lane == k, m, outv_ref[0])
        outi_ref[0] = jnp.where(lane == k, f, outi_ref[0])
        row = lax.shift_right_logical(f, 8)
        col = jnp.bitwise_and(f, 255)
        srow = s_ref[pl.ds(row, 1), :]
        s_ref[pl.ds(row, 1), :] = jnp.where(colio == col, -1.0, srow)
        g = lax.shift_right_logical(row, 4)
        blk = s_ref[pl.ds(g * 16, 16), :]
        bfi = bio + g * 4096
        bm = jnp.max(blk, axis=0, keepdims=True)
        bi = jnp.min(jnp.where(blk == bm, bfi, _BIG), axis=0, keepdims=True)
        l1v_ref[pl.ds(g, 1), :] = bm
        l1i_ref[pl.ds(g, 1), :] = bi
        g2 = lax.shift_right_logical(g, 3)
        b2v = l1v_ref[pl.ds(g2 * 8, 8), :]
        b2i = l1i_ref[pl.ds(g2 * 8, 8), :]
        b2m = jnp.max(b2v, axis=0, keepdims=True)
        b2mi = jnp.min(jnp.where(b2v == b2m, b2i, _BIG), axis=0,
                       keepdims=True)
        l2v_ref[pl.ds(g2, 1), :] = b2m
        l2i_ref[pl.ds(g2, 1), :] = b2mi
        return 0

    lax.fori_loop(0, _TOP_K, body, 0)


def _topk_tc(heat, interpret=False):
    B, C, H, W = heat.shape
    return pl.pallas_call(
        _topk_body,
        out_shape=[jax.ShapeDtypeStruct((B, 1, _KP), jnp.float32),
                   jax.ShapeDtypeStruct((B, 1, _KP), jnp.int32)],
        grid=(B,),
        in_specs=[pl.BlockSpec((1, C, H, W), lambda b: (b, 0, 0, 0))],
        out_specs=[pl.BlockSpec((1, 1, _KP), lambda b: (b, 0, 0)),
                   pl.BlockSpec((1, 1, _KP), lambda b: (b, 0, 0))],
        scratch_shapes=[
            pltpu.VMEM((2048, 256), jnp.float32),
            pltpu.VMEM((128, 256), jnp.float32),
            pltpu.VMEM((128, 256), jnp.int32),
            pltpu.VMEM((16, 256), jnp.float32),
            pltpu.VMEM((16, 256), jnp.int32),
        ],
        interpret=interpret,
    )(heat)


def _sc_gather_fn(B):
    import functools
    from jax.experimental.pallas import tpu_sc as plsc

    HW = 65536
    ANG = np.float32(2.0 * np.pi / _NUM_DIR_BINS)
    NPTS = B * _KP
    PW = NPTS // 32
    mesh = plsc.VectorSubcoreMesh(core_axis_name="c", subcore_axis_name="s")
    nj = PW // 16

    @functools.partial(
        pl.kernel,
        out_type=jax.ShapeDtypeStruct((9 * NPTS,), jnp.float32),
        mesh=mesh,
        scratch_types=[
            pltpu.VMEM((PW,), jnp.int32),
            pltpu.VMEM((PW,), jnp.float32),
            pltpu.VMEM((2 * PW,), jnp.int32),
            pltpu.VMEM((2 * PW,), jnp.int32),
            pltpu.VMEM((12 * PW,), jnp.int32),
            pltpu.VMEM((PW,), jnp.int32),
            pltpu.VMEM((PW,), jnp.int32),
            pltpu.VMEM((2 * PW,), jnp.float32),
            pltpu.VMEM((2 * PW,), jnp.float32),
            pltpu.VMEM((12 * PW,), jnp.float32),
            pltpu.VMEM((PW,), jnp.float32),
            pltpu.VMEM((PW,), jnp.float32),
            pltpu.VMEM((9, PW), jnp.float32),
            pltpu.SemaphoreType.DMA,
        ],
    )
    def k(inds_hbm, scores_hbm, wh_hbm, off_hbm, yc_hbm, yr_hbm, vel_hbm,
          out_hbm, idx_v, sc_v, iwh, ioff, iyc, iyr, ivl,
          gwh, goff, gyc, gyr, gvl, outb, sem):
        wid = lax.axis_index("s") * 2 + lax.axis_index("c")
        base = wid * PW
        b = base // _KP
        pltpu.sync_copy(inds_hbm.at[pl.ds(base, PW)], idx_v)
        pltpu.sync_copy(scores_hbm.at[pl.ds(base, PW)], sc_v)
        for j in range(nj):
            sl = pl.ds(j * 16, 16)
            f = idx_v[sl]
            sp = jnp.bitwise_and(f, HW - 1)
            iyr[sl] = b * HW + sp
            ivl[sl] = b * HW + sp
            for c in range(2):
                iwh[pl.ds(c * PW + j * 16, 16)] = (b * 2 + c) * HW + sp
                ioff[pl.ds(c * PW + j * 16, 16)] = (b * 2 + c) * HW + sp
            for c in range(12):
                iyc[pl.ds(c * PW + j * 16, 16)] = (b * 12 + c) * HW + sp
        cps = [pltpu.async_copy(wh_hbm.at[iwh], gwh, sem),
               pltpu.async_copy(off_hbm.at[ioff], goff, sem),
               pltpu.async_copy(yc_hbm.at[iyc], gyc, sem),
               pltpu.async_copy(yr_hbm.at[iyr], gyr, sem),
               pltpu.async_copy(vel_hbm.at[ivl], gvl, sem)]
        for cp in cps:
            cp.wait()
        for j in range(nj):
            sl = pl.ds(j * 16, 16)
            f = idx_v[sl]
            sp = jnp.bitwise_and(f, HW - 1)
            xx = jnp.bitwise_and(sp, 255).astype(jnp.float32)
            yy = lax.shift_right_logical(sp, 8).astype(jnp.float32)
            outb[0, sl] = (xx + goff[pl.ds(j * 16, 16)]) * _PPM
            outb[1, sl] = (yy + goff[pl.ds(PW + j * 16, 16)]) * _PPM
            outb[2, sl] = gwh[pl.ds(j * 16, 16)] * _PPM
            outb[3, sl] = gwh[pl.ds(PW + j * 16, 16)] * _PPM
            bestv = gyc[pl.ds(j * 16, 16)]
            bestc = jnp.zeros((16,), jnp.int32)
            for c in range(1, 12):
                v = gyc[pl.ds(c * PW + j * 16, 16)]
                better = v > bestv
                bestv = jnp.where(better, v, bestv)
                bestc = jnp.where(better, jnp.int32(c), bestc)
            outb[4, sl] = bestc.astype(jnp.float32) * ANG + gyr[sl]
            outb[5, sl] = gvl[sl]
            outb[6, sl] = jnp.zeros((16,), jnp.float32)
            outb[7, sl] = lax.shift_right_logical(f, 16).astype(jnp.float32)
            outb[8, sl] = sc_v[sl]
        for comp in range(9):
            pltpu.sync_copy(outb.at[comp],
                            out_hbm.at[pl.ds(comp * NPTS + base, PW)])

    return k


def kernel(center_heatmap_pred, wh_pred, offset_pred, yaw_class_pred,
           yaw_res_pred, velocity_pred):
    B, C, H, W = center_heatmap_pred.shape
    scores_p, inds_p = _topk_tc(center_heatmap_pred)
    out9 = _sc_gather_fn(B)(
        inds_p.reshape(B * _KP), scores_p.reshape(B * _KP),
        wh_pred.reshape(-1), offset_pred.reshape(-1),
        yaw_class_pred.reshape(-1), yaw_res_pred.reshape(-1),
        velocity_pred.reshape(-1))
    return jnp.transpose(out9.reshape(9, B, _KP), (1, 2, 0))[:, :_TOP_K, :]
